# baseline (speedup 1.0000x reference)
import functools

import jax
import jax.numpy as jnp
import numpy as np
from jax import lax
from jax.experimental import pallas as pl
from jax.experimental.pallas import tpu as pltpu
from jax.experimental.pallas import tpu_sc as plsc

VOCAB = 1000000
EMBED = 64
HIDDEN = 512
NUM_CLASSES = 128
BATCH = 4096
SEQ = 200

_LANES = 128
_CHUNK = 128


def _prep_body(t_ref, o_ref):
    o_ref[:, :EMBED] = t_ref[...].astype(jnp.bfloat16)
    o_ref[:, EMBED:] = jnp.zeros_like(o_ref[:, EMBED:])


def _prep_table(table):
    blk = 8000
    return pl.pallas_call(
        _prep_body,
        grid=(VOCAB // blk,),
        in_specs=[pl.BlockSpec((blk, EMBED), lambda i: (i, 0))],
        out_specs=pl.BlockSpec((blk, _LANES), lambda i: (i, 0)),
        out_shape=jax.ShapeDtypeStruct((VOCAB, _LANES), jnp.bfloat16),
    )(table)


def _make_sc_pool():
    info = plsc.get_sparse_core_info()
    nc, ns = info.num_cores, info.num_subcores
    nw = nc * ns
    rows_per_w = BATCH // nw
    idx_per_w = rows_per_w * SEQ
    chunks_per_w = idx_per_w // _CHUNK

    mesh = plsc.VectorSubcoreMesh(core_axis_name="c", subcore_axis_name="s")

    @functools.partial(
        pl.kernel,
        mesh=mesh,
        compiler_params=pltpu.CompilerParams(
            use_tc_tiling_on_sc=False, needs_layout_passes=False),
        out_type=jax.ShapeDtypeStruct((BATCH, EMBED), jnp.float32),
        scratch_types=[
            pltpu.VMEM((chunks_per_w, _CHUNK), jnp.int32),
            pltpu.VMEM((_CHUNK, _LANES), jnp.bfloat16),
            pltpu.VMEM((_CHUNK, _LANES), jnp.bfloat16),
            pltpu.VMEM((rows_per_w, EMBED), jnp.float32),
            pltpu.SemaphoreType.DMA,
            pltpu.SemaphoreType.DMA,
        ],
    )
    def sc_pool(x_hbm, tbl_hbm, out_hbm, idx_v, rows0, rows1, out_v,
                sem0, sem1):
        wid = lax.axis_index("s") * nc + lax.axis_index("c")

        pltpu.sync_copy(x_hbm.at[pl.ds(wid * chunks_per_w, chunks_per_w)],
                        idx_v)

        zero = jnp.zeros((16,), jnp.float32)

        def fire(c, buf, sem):
            pltpu.async_copy(tbl_hbm.at[idx_v.at[c]], buf, sem)

        def wait(buf, sem):
            pltpu.make_async_copy(tbl_hbm.at[idx_v.at[0]], buf, sem).wait()

        def accum_range(buf, lo, hi, acc):
            def body(r, a):
                new = []
                for t in range(2):
                    w = plsc.bitcast(buf[r, pl.ds(32 * t, 32)], jnp.int32)
                    even = plsc.bitcast(w << 16, jnp.float32)
                    odd = plsc.bitcast(w, jnp.float32)
                    new.append(a[2 * t] + even)
                    new.append(a[2 * t + 1] + odd)
                return tuple(new)
            return lax.fori_loop(lo, hi, body, acc)

        def process(c, buf, acc):
            pos0 = c * _CHUNK
            q0 = pos0 // SEQ
            s = (q0 + 1) * SEQ - pos0
            sl = jnp.minimum(s, _CHUNK)
            acc = accum_range(buf, 0, sl, acc)

            def flush_and_rest():
                for g in range(4):
                    out_v[q0, pl.ds(16 * g, 16)] = acc[g]
                return accum_range(buf, sl, _CHUNK, (zero,) * 4)

            return lax.cond(s <= _CHUNK, flush_and_rest, lambda: acc)

        fire(0, rows0, sem0)

        def pair_body(p, acc):
            fire(2 * p + 1, rows1, sem1)
            wait(rows0, sem0)
            acc = process(2 * p, rows0, acc)

            @pl.when(p < chunks_per_w // 2 - 1)
            def _():
                fire(2 * p + 2, rows0, sem0)

            wait(rows1, sem1)
            return process(2 * p + 1, rows1, acc)

        lax.fori_loop(0, chunks_per_w // 2, pair_body, (zero,) * 4)

        pltpu.sync_copy(out_v, out_hbm.at[pl.ds(wid * rows_per_w, rows_per_w)])

    return sc_pool


_sc_pool = None


def _mlp_body(p_ref, w1_ref, b1_ref, w2_ref, b2_ref, o_ref):
    h = jnp.dot(p_ref[...], w1_ref[...], preferred_element_type=jnp.float32)
    h = jnp.maximum(h + b1_ref[...], 0.0)
    o = jnp.dot(h, w2_ref[...], preferred_element_type=jnp.float32)
    o_ref[...] = o + b2_ref[...]


def _mlp(pooled, W1p, b1, W2, b2):
    blk = 512
    return pl.pallas_call(
        _mlp_body,
        grid=(BATCH // blk,),
        in_specs=[
            pl.BlockSpec((blk, EMBED), lambda i: (i, 0)),
            pl.BlockSpec((EMBED, HIDDEN), lambda i: (0, 0)),
            pl.BlockSpec((1, HIDDEN), lambda i: (0, 0)),
            pl.BlockSpec((HIDDEN, NUM_CLASSES), lambda i: (0, 0)),
            pl.BlockSpec((1, NUM_CLASSES), lambda i: (0, 0)),
        ],
        out_specs=pl.BlockSpec((blk, NUM_CLASSES), lambda i: (i, 0)),
        out_shape=jax.ShapeDtypeStruct((BATCH, NUM_CLASSES), jnp.float32),
    )(pooled, W1p, b1.reshape(1, HIDDEN), W2, b2.reshape(1, NUM_CLASSES))


_J = np.arange(EMBED)
_COL = 32 * (_J // 32) + 2 * (_J % 16) + ((_J // 16) % 2)


def _prep_w1(W1):
    return W1[_COL, :] * (1.0 / SEQ)


def kernel(x, table, W1, b1, W2, b2):
    global _sc_pool
    if _sc_pool is None:
        _sc_pool = _make_sc_pool()
    tblp = _prep_table(table)
    x_chunks = x.astype(jnp.int32).reshape(BATCH * SEQ // _CHUNK, _CHUNK)
    pooled = _sc_pool(x_chunks, tblp)
    return _mlp(pooled, _prep_w1(W1), b1, W2, b2)


# speedup vs baseline: 2.2146x; 2.2146x over previous
import functools

import jax
import jax.numpy as jnp
from jax import lax
from jax.experimental import pallas as pl
from jax.experimental.pallas import tpu as pltpu
from jax.experimental.pallas import tpu_sc as plsc

VOCAB = 1000000
EMBED = 64
HIDDEN = 512
NUM_CLASSES = 128
BATCH = 4096
SEQ = 200

_CA = 104
_CB = SEQ - _CA
_EG = EMBED // 16


def _make_sc_pool():
    info = plsc.get_sparse_core_info()
    nc, ns = info.num_cores, info.num_subcores
    nw = nc * ns
    rows_per_w = BATCH // nw

    mesh = plsc.VectorSubcoreMesh(core_axis_name="c", subcore_axis_name="s")

    @functools.partial(
        pl.kernel,
        mesh=mesh,
        compiler_params=pltpu.CompilerParams(use_tc_tiling_on_sc=False),
        out_type=jax.ShapeDtypeStruct((BATCH, EMBED), jnp.float32),
        scratch_types=[
            pltpu.VMEM((rows_per_w, SEQ), jnp.int32),
            pltpu.VMEM((_CA, EMBED), jnp.float32),
            pltpu.VMEM((_CA, EMBED), jnp.float32),
            pltpu.VMEM((_CB, EMBED), jnp.float32),
            pltpu.VMEM((_CB, EMBED), jnp.float32),
            pltpu.VMEM((rows_per_w, EMBED), jnp.float32),
            pltpu.SemaphoreType.DMA,
            pltpu.SemaphoreType.DMA,
            pltpu.SemaphoreType.DMA,
            pltpu.SemaphoreType.DMA,
        ],
    )
    def sc_pool(x_hbm, table_hbm, out_hbm, idx_v, bufa0, bufa1, bufb0, bufb1,
                out_v, sema0, sema1, semb0, semb1):
        wid = lax.axis_index("s") * nc + lax.axis_index("c")
        row0 = wid * rows_per_w

        pltpu.sync_copy(x_hbm.at[pl.ds(row0, rows_per_w)], idx_v)

        def fire(i, buf, sem, off, n):
            pltpu.async_copy(
                table_hbm.at[idx_v.at[i, pl.ds(off, n)]], buf, sem)

        def wait(buf, sem):
            pltpu.make_async_copy(
                table_hbm.at[idx_v.at[0, pl.ds(0, buf.shape[0])]], buf,
                sem).wait()

        def sum_chunk(buf, n, acc):
            def body(r, a):
                return tuple(
                    a[g] + buf[r, pl.ds(16 * g, 16)] for g in range(_EG)
                )
            return lax.fori_loop(0, n, body, acc, unroll=4)

        zero = jnp.zeros((16,), jnp.float32)

        fire(0, bufa0, sema0, 0, _CA)
        fire(0, bufb0, semb0, _CA, _CB)

        def do_row(i, bufa, sema, bufb, semb, fire_next, next_a, next_sa,
                   next_b, next_sb):
            @pl.when(fire_next)
            def _():
                fire(i + 1, next_a, next_sa, 0, _CA)
                fire(i + 1, next_b, next_sb, _CA, _CB)

            wait(bufa, sema)
            acc = sum_chunk(bufa, _CA, (zero,) * _EG)
            wait(bufb, semb)
            acc = sum_chunk(bufb, _CB, acc)
            for g in range(_EG):
                out_v[i, pl.ds(16 * g, 16)] = acc[g]

        def pair_body(k, _):
            i = 2 * k
            do_row(i, bufa0, sema0, bufb0, semb0, True,
                   bufa1, sema1, bufb1, semb1)
            do_row(i + 1, bufa1, sema1, bufb1, semb1,
                   i + 2 < rows_per_w, bufa0, sema0, bufb0, semb0)
            return 0

        lax.fori_loop(0, rows_per_w // 2, pair_body, 0)

        pltpu.sync_copy(out_v, out_hbm.at[pl.ds(row0, rows_per_w)])

    return sc_pool


_sc_pool = None


def _mlp_body(p_ref, w1_ref, b1_ref, w2_ref, b2_ref, o_ref):
    p = p_ref[...] * (1.0 / SEQ)
    h = jnp.dot(p, w1_ref[...], preferred_element_type=jnp.float32)
    h = jnp.maximum(h + b1_ref[...], 0.0)
    o = jnp.dot(h, w2_ref[...], preferred_element_type=jnp.float32)
    o_ref[...] = o + b2_ref[...]


def _mlp(pooled, W1, b1, W2, b2):
    blk = 512
    return pl.pallas_call(
        _mlp_body,
        grid=(BATCH // blk,),
        in_specs=[
            pl.BlockSpec((blk, EMBED), lambda i: (i, 0)),
            pl.BlockSpec((EMBED, HIDDEN), lambda i: (0, 0)),
            pl.BlockSpec((1, HIDDEN), lambda i: (0, 0)),
            pl.BlockSpec((HIDDEN, NUM_CLASSES), lambda i: (0, 0)),
            pl.BlockSpec((1, NUM_CLASSES), lambda i: (0, 0)),
        ],
        out_specs=pl.BlockSpec((blk, NUM_CLASSES), lambda i: (i, 0)),
        out_shape=jax.ShapeDtypeStruct((BATCH, NUM_CLASSES), jnp.float32),
    )(pooled, W1, b1.reshape(1, HIDDEN), W2, b2.reshape(1, NUM_CLASSES))


def kernel(x, table, W1, b1, W2, b2):
    global _sc_pool
    if _sc_pool is None:
        _sc_pool = _make_sc_pool()
    pooled = _sc_pool(x.astype(jnp.int32), table)
    return _mlp(pooled, W1, b1, W2, b2)
